# baseline (speedup 1.0000x reference)
import jax
import jax.numpy as jnp
from jax import lax
from jax.experimental import pallas as pl
from jax.experimental.pallas import tpu as pltpu
from jax.experimental.pallas import tpu_sc as plsc

_EMB = 64
_NC = 2
_NS = 16
_NW = _NC * _NS

_BB = 128
_L = 16
_D = 4


def _gather_body(idx_hbm, table_hbm, out_hbm,
                 idx0, idx1, idx2, idx3,
                 rows0, rows1, rows2, rows3, st0, st1,
                 si0, si1, si2, si3,
                 sg0, sg1, sg2, sg3, so0, so1):
  n_s1, n_b1, n_s2, _ = idx_hbm.shape
  seq = n_s1 * n_s2
  n_units = seq * n_b1
  per_w = n_units // _NW
  wid = lax.axis_index("s") * _NC + lax.axis_index("c")
  base = wid * per_w

  idxs = (idx0, idx1, idx2, idx3)
  rows = (rows0, rows1, rows2, rows3)
  stg = (st0, st1)
  sem_i = (si0, si1, si2, si3)
  sem_g = (sg0, sg1, sg2, sg3)
  sem_o = (so0, so1)

  iota = lax.iota(jnp.int32, _L)
  row_ids = [iota + _L * k for k in range(_BB // _L)]

  def unit_coords(u):
    s = u // n_b1
    b1 = u % n_b1
    return s // n_s2, s % n_s2, b1, s

  def load_idx(u, s):
    s1, s2, b1, _ = unit_coords(u)
    pltpu.async_copy(idx_hbm.at[s1, b1, s2], idxs[s], sem_i[s])

  def wait_idx(s):
    pltpu.make_async_copy(idx_hbm.at[0, 0, 0], idxs[s], sem_i[s]).wait()

  def fire_gather(s):
    pltpu.async_copy(table_hbm.at[idxs[s]], rows[s], sem_g[s])

  def wait_gather(s):
    pltpu.make_async_copy(
        table_hbm.at[pl.ds(0, _BB)], rows[s], sem_g[s]).wait()

  def fire_stores(u, s):
    _, _, b1, sq = unit_coords(u)
    for e1 in range(_EMB // 8):
      pltpu.async_copy(
          stg[s].at[pl.ds(8 * e1, 8)], out_hbm.at[sq, e1, b1], sem_o[s])

  def wait_stores(s):
    for e1 in range(_EMB // 8):
      pltpu.make_async_copy(
          stg[s].at[pl.ds(8 * e1, 8)], out_hbm.at[0, e1, 0], sem_o[s]).wait()

  def transpose(rs, ts):
    def body(d, carry):
      perm = lax.rem(iota + d, _L)
      for m in range(_EMB // _L):
        colv = perm + _L * m
        for k in range(_BB // _L):
          vals = plsc.load_gather(rows[rs], [row_ids[k], colv])
          plsc.store_scatter(stg[ts], [colv, row_ids[k]], vals)
      return carry
    lax.fori_loop(0, _L, body, 0)

  def step(i, rs, ts):
    wait_gather(rs)

    @pl.when(i + _D < per_w)
    def _():
      load_idx(base + i + _D, rs)

    @pl.when(i + _D - 1 < per_w)
    def _():
      wait_idx((rs + _D - 1) % _D)
      fire_gather((rs + _D - 1) % _D)

    @pl.when(i >= 2)
    def _():
      wait_stores(ts)

    transpose(rs, ts)
    fire_stores(base + i, ts)

  for u in range(_D):
    load_idx(base + u, u)
  for u in range(_D - 1):
    wait_idx(u)
    fire_gather(u)

  def quad(k, carry):
    i0 = _D * k
    for j in range(_D):
      step(i0 + j, j, j % 2)
    return carry

  lax.fori_loop(0, per_w // _D, quad, 0)

  wait_stores(0)
  wait_stores(1)


import functools


@functools.partial(jax.jit, static_argnums=(2, 3))
def _embed_lookup(idx4d, table, b, s):
  run = pl.kernel(
      _gather_body,
      out_type=jax.ShapeDtypeStruct(
          (s, _EMB // 8, b // _BB, 8, _BB), jnp.float32),
      mesh=plsc.VectorSubcoreMesh(
          core_axis_name="c", subcore_axis_name="s",
          num_cores=_NC, num_subcores=_NS,
      ),
      scratch_types=(
          [pltpu.VMEM((_BB,), jnp.int32)] * 4
          + [pltpu.VMEM((_BB, _EMB), jnp.float32)] * 4
          + [pltpu.VMEM((_EMB, _BB), jnp.float32)] * 2
          + [pltpu.SemaphoreType.DMA] * 10
      ),
      compiler_params=pltpu.CompilerParams(
          use_tc_tiling_on_sc=False, needs_layout_passes=False),
  )
  return run(idx4d, table)


def kernel(input_tensor, token_table):
  b, s = input_tensor.shape
  idx = input_tensor.astype(jnp.int32)
  idx4d = idx.reshape(b // _BB, _BB, s // 8, 8).transpose(2, 0, 3, 1)
  out5d = _embed_lookup(idx4d, token_table, b, s)
  return out5d.transpose(2, 4, 0, 1, 3).reshape(b, s, _EMB)
